# baseline (speedup 1.0000x reference)
import functools

import jax
import jax.numpy as jnp
from jax import lax
from jax.experimental import pallas as pl
from jax.experimental.pallas import tpu as pltpu
from jax.experimental.pallas import tpu_sc as plsc

_NPOINT = 512
_RADII = (0.1, 0.2, 0.4)
_NSAMPLES = (16, 32, 64)
_TABLE_D = 128


_SBLK = 128


def _select_body(q_ref, p_ref, e1_ref, e2_ref, e3_ref,
                 s1_ref, s2_ref, s3_ref):
    N = p_ref.shape[2]
    q = q_ref[0]
    p = p_ref[0]
    qp = jax.lax.dot_general(q, p, (((1,), (0,)), ((), ())),
                             preferred_element_type=jnp.float32)
    q2 = jnp.sum(q * q, axis=1, keepdims=True)
    p2 = jnp.sum(p * p, axis=0, keepdims=True)
    d = -2.0 * qp + q2 + p2

    masks = [(d <= r * r).astype(jnp.float32) for r in _RADII]
    m_all = jnp.concatenate(masks, axis=0)

    it = jax.lax.broadcasted_iota(jnp.int32, (128, 128), 0)
    jt = jax.lax.broadcasted_iota(jnp.int32, (128, 128), 1)
    T = (it <= jt).astype(jnp.float32)

    carry = jnp.zeros((3 * _SBLK, 1), jnp.float32)
    smax = [jnp.full((_SBLK, 1), -1, jnp.int32) for _ in range(3)]
    jl0 = jax.lax.broadcasted_iota(jnp.int32, (_SBLK, 128), 1)
    for g in range(N // 128):
        seg = m_all[:, g * 128:(g + 1) * 128]
        loc = jax.lax.dot_general(seg, T, (((1,), (0,)), ((), ())),
                                  preferred_element_type=jnp.float32)
        C = loc + carry
        carry = carry + loc[:, 127:128]
        byte = g % 4
        woff = (g // 4) * 128
        for ri, (e_ref, K) in enumerate(
                zip((e1_ref, e2_ref, e3_ref), _NSAMPLES)):
            Cg = C[ri * _SBLK:(ri + 1) * _SBLK]
            mg = m_all[ri * _SBLK:(ri + 1) * _SBLK, g * 128:(g + 1) * 128]
            e = jnp.where(mg > 0.0,
                          jnp.minimum(Cg, float(K + 1)), 0.0).astype(jnp.int32)
            cand = jnp.max(jnp.where(e == K, jl0 + g * 128, -1),
                           axis=1, keepdims=True)
            smax[ri] = jnp.maximum(smax[ri], cand)
            word = e << (8 * byte)
            if byte == 0:
                e_ref[0, :, woff:woff + 128] = word
            else:
                e_ref[0, :, woff:woff + 128] = e_ref[0, :, woff:woff + 128] | word
    for ri, s_ref in enumerate((s1_ref, s2_ref, s3_ref)):
        s_ref[0] = jnp.where(smax[ri] < 0, N, smax[ri])


def _select(new_xyz, xyz):
    B, _, N = xyz.shape
    S = _NPOINT
    grid = (B, S // _SBLK)
    outs = pl.pallas_call(
        _select_body,
        grid=grid,
        in_specs=[
            pl.BlockSpec((1, _SBLK, 3), lambda b, s: (b, s, 0)),
            pl.BlockSpec((1, 3, N), lambda b, s: (b, 0, 0)),
        ],
        out_specs=([pl.BlockSpec((1, _SBLK, N // 4), lambda b, s: (b, s, 0))
                    for _ in range(3)] +
                   [pl.BlockSpec((1, _SBLK, 1), lambda b, s: (b, s, 0))
                    for _ in range(3)]),
        out_shape=([jax.ShapeDtypeStruct((B, S, N // 4), jnp.int32)
                    for _ in range(3)] +
                   [jax.ShapeDtypeStruct((B, S, 1), jnp.int32)
                    for _ in range(3)]),
    )(new_xyz, xyz)
    return ([o.reshape(B * S, N // 4) for o in outs[:3]],
            [o.reshape(B * S) for o in outs[3:]])


def _sc_extract_gather(table, e_packed, stopw, K, N):
    RQ, NWRD = e_packed.shape
    NC, NS = 2, 16
    rows_per_w = RQ // (NC * NS)
    mesh = plsc.VectorSubcoreMesh(core_axis_name="c", subcore_axis_name="s")

    @functools.partial(
        pl.kernel, mesh=mesh,
        compiler_params=pltpu.CompilerParams(needs_layout_passes=False),
        out_type=jax.ShapeDtypeStruct((RQ * K, 128), jnp.float32),
        scratch_types=[
            pltpu.VMEM((NWRD,), jnp.int32),
            pltpu.VMEM((8, 128), jnp.int32),
            pltpu.VMEM((K,), jnp.int32),
            pltpu.VMEM((K, 128), jnp.float32),
            pltpu.VMEM((rows_per_w,), jnp.int32),
            pltpu.SemaphoreType.DMA,
        ],
    )
    def k(table_hbm, e_hbm, stop_hbm, out_hbm, e_v, idx_v, gidx_v, rows_v,
          stop_v, sem):
        wid = lax.axis_index("s") * NC + lax.axis_index("c")
        base = wid * rows_per_w
        lane = jax.lax.iota(jnp.int32, 16)
        zero16 = jnp.zeros((16,), jnp.int32)
        sentinel = jnp.full((16,), N, jnp.int32)
        pltpu.sync_copy(stop_hbm.at[pl.ds(base, rows_per_w)], stop_v)

        def row_body(i, _):
            row = base + i
            pltpu.sync_copy(e_hbm.at[row], e_v)
            for g in range(K // 16):
                idx_v[0, pl.ds(g * 16, 16)] = sentinel

            sv = stop_v[pl.ds((i // 16) * 16, 16)]
            sj = jnp.max(jnp.where(lane == (i % 16), sv, -1), axis=0)
            c_stop = jnp.minimum(NWRD // 16, (sj >> 9) * 8 + 8)

            def wchunk(c, _c):
                wv = e_v[pl.ds(c * 16, 16)]
                jb = 512 * (c // 8) + (c % 8) * 16
                for b in range(4):
                    eb = (wv >> (8 * b)) & 0xFF
                    valid = (eb > 0) & (eb <= K)
                    plsc.store_scatter(idx_v, [zero16, eb - 1],
                                       lane + (jb + 128 * b), mask=valid)
                return _c

            lax.fori_loop(0, c_stop, wchunk, 0)

            first = jnp.broadcast_to(
                jnp.min(idx_v[0, pl.ds(0, 16)], axis=0), (16,))
            boff = jnp.full((16,), (row // _NPOINT) * N, jnp.int32)
            for g in range(K // 16):
                v = idx_v[0, pl.ds(g * 16, 16)]
                gidx_v[pl.ds(g * 16, 16)] = (
                    jnp.where(v == N, first, v) + boff)
            pltpu.async_copy(table_hbm.at[gidx_v], rows_v, sem).wait()
            pltpu.sync_copy(rows_v, out_hbm.at[pl.ds(row * K, K)])
            return _

        lax.fori_loop(0, rows_per_w, row_body, 0)

    return k(table, e_packed, stopw)


def _fps_body(xyz_ref, out_ref, nxyz_ref):
    B, _, N = xyz_ref.shape
    x = xyz_ref[:, 0, :]
    y = xyz_ref[:, 1, :]
    z = xyz_ref[:, 2, :]
    iota = jax.lax.broadcasted_iota(jnp.int32, (B, N), 1)

    def step(i, carry):
        dist, far = carry
        out_ref[pl.ds(i, 1), :] = far.T
        sel = iota == far
        cx = jnp.sum(jnp.where(sel, x, 0.0), axis=1, keepdims=True)
        cy = jnp.sum(jnp.where(sel, y, 0.0), axis=1, keepdims=True)
        cz = jnp.sum(jnp.where(sel, z, 0.0), axis=1, keepdims=True)
        nxyz_ref[pl.ds(i, 1), :, :] = jnp.concatenate(
            [cx, cy, cz], axis=1)[None, :, :]
        dx = x - cx
        dy = y - cy
        dz = z - cz
        d = dx * dx + dy * dy + dz * dz
        dist = jnp.minimum(dist, d)
        m = jnp.max(dist, axis=1, keepdims=True)
        far_new = jnp.min(jnp.where(dist == m, iota, N), axis=1, keepdims=True)
        return dist, far_new.astype(jnp.int32)

    dist0 = jnp.full((B, N), 1e10, dtype=jnp.float32)
    far0 = jnp.zeros((B, 1), dtype=jnp.int32)
    jax.lax.fori_loop(0, out_ref.shape[0], step, (dist0, far0))


def _fps(xyz):
    B, _, N = xyz.shape
    out, nxyz = pl.pallas_call(
        _fps_body,
        out_shape=[jax.ShapeDtypeStruct((_NPOINT, B), jnp.int32),
                   jax.ShapeDtypeStruct((_NPOINT, B, 3), jnp.float32)],
        in_specs=[pl.BlockSpec(memory_space=pltpu.MemorySpace.VMEM)],
        out_specs=[pl.BlockSpec(memory_space=pltpu.MemorySpace.VMEM),
                   pl.BlockSpec(memory_space=pltpu.MemorySpace.VMEM)],
    )(xyz)
    return out.T, jnp.transpose(nxyz, (1, 0, 2))


def _index_points(points, idx):
    return jax.vmap(lambda p, i: p[i])(points, idx)


def kernel(xyz, points, params):
    B, _, N = xyz.shape
    S = _NPOINT
    xyz_t = jnp.transpose(xyz, (0, 2, 1))
    pts_t = jnp.transpose(points, (0, 2, 1))

    _, new_xyz = _fps(xyz)

    table = jnp.concatenate(
        [pts_t, xyz_t, jnp.zeros((B, N, _TABLE_D - 67), jnp.float32)],
        axis=-1).reshape(B * N, _TABLE_D)

    e_packed, stops = _select(new_xyz, xyz)

    outs = []
    for ri, (r, K) in enumerate(zip(_RADII, _NSAMPLES)):
        g_rows = _sc_extract_gather(table, e_packed[ri], stops[ri], K, N)
        g_rows = g_rows.reshape(B, S, K, _TABLE_D)
        g_pts = g_rows[..., :64]
        g_xyz = g_rows[..., 64:67] - new_xyz[:, :, None, :]
        g = jnp.concatenate([g_pts, g_xyz], axis=-1)
        g = jnp.transpose(g, (0, 3, 2, 1))
        for layer in params[len(outs)]:
            g = jnp.einsum('oc,bcks->boks', layer["W"], g) + layer["b"][None, :, None, None]
            mean = jnp.mean(g, axis=(0, 2, 3), keepdims=True)
            var = jnp.var(g, axis=(0, 2, 3), keepdims=True)
            g = (g - mean) / jnp.sqrt(var + 1e-5)
            g = g * layer["gamma"][None, :, None, None] + layer["beta"][None, :, None, None]
            g = jax.nn.relu(g)
        outs.append(jnp.max(g, axis=2))

    return (jnp.transpose(new_xyz, (0, 2, 1)), jnp.concatenate(outs, axis=1))


# speedup vs baseline: 1.1213x; 1.1213x over previous
import functools

import jax
import jax.numpy as jnp
from jax import lax
from jax.experimental import pallas as pl
from jax.experimental.pallas import tpu as pltpu
from jax.experimental.pallas import tpu_sc as plsc

_NPOINT = 512
_RADII = (0.1, 0.2, 0.4)
_NSAMPLES = (16, 32, 64)
_TABLE_D = 128


_SBLK = 128


def _select_body(q_ref, p_ref, e1_ref, e2_ref, e3_ref,
                 s1_ref, s2_ref, s3_ref):
    N = p_ref.shape[2]
    q = q_ref[0]
    p = p_ref[0]
    qp = jax.lax.dot_general(q, p, (((1,), (0,)), ((), ())),
                             preferred_element_type=jnp.float32)
    q2 = jnp.sum(q * q, axis=1, keepdims=True)
    p2 = jnp.sum(p * p, axis=0, keepdims=True)
    d = -2.0 * qp + q2 + p2

    masks = [(d <= r * r).astype(jnp.float32) for r in _RADII]
    m_all = jnp.concatenate(masks, axis=0)

    it = jax.lax.broadcasted_iota(jnp.int32, (128, 128), 0)
    jt = jax.lax.broadcasted_iota(jnp.int32, (128, 128), 1)
    T = (it <= jt).astype(jnp.float32)

    carry = jnp.zeros((3 * _SBLK, 1), jnp.float32)
    smax = [jnp.full((_SBLK, 1), -1, jnp.int32) for _ in range(3)]
    jl0 = jax.lax.broadcasted_iota(jnp.int32, (_SBLK, 128), 1)
    for g in range(N // 128):
        seg = m_all[:, g * 128:(g + 1) * 128]
        loc = jax.lax.dot_general(seg, T, (((1,), (0,)), ((), ())),
                                  preferred_element_type=jnp.float32)
        C = loc + carry
        carry = carry + loc[:, 127:128]
        byte = g % 4
        woff = (g // 4) * 128
        for ri, (e_ref, K) in enumerate(
                zip((e1_ref, e2_ref, e3_ref), _NSAMPLES)):
            Cg = C[ri * _SBLK:(ri + 1) * _SBLK]
            mg = m_all[ri * _SBLK:(ri + 1) * _SBLK, g * 128:(g + 1) * 128]
            e = jnp.where(mg > 0.0,
                          jnp.minimum(Cg, float(K + 1)), 0.0).astype(jnp.int32)
            cand = jnp.max(jnp.where(e == K, jl0 + g * 128, -1),
                           axis=1, keepdims=True)
            smax[ri] = jnp.maximum(smax[ri], cand)
            word = e << (8 * byte)
            if byte == 0:
                e_ref[0, :, woff:woff + 128] = word
            else:
                e_ref[0, :, woff:woff + 128] = e_ref[0, :, woff:woff + 128] | word
    for ri, s_ref in enumerate((s1_ref, s2_ref, s3_ref)):
        s_ref[0] = jnp.where(smax[ri] < 0, N, smax[ri])


def _select(new_xyz, xyz):
    B, _, N = xyz.shape
    S = _NPOINT
    grid = (B, S // _SBLK)
    outs = pl.pallas_call(
        _select_body,
        grid=grid,
        in_specs=[
            pl.BlockSpec((1, _SBLK, 3), lambda b, s: (b, s, 0)),
            pl.BlockSpec((1, 3, N), lambda b, s: (b, 0, 0)),
        ],
        out_specs=([pl.BlockSpec((1, _SBLK, N // 4), lambda b, s: (b, s, 0))
                    for _ in range(3)] +
                   [pl.BlockSpec((1, _SBLK, 1), lambda b, s: (b, s, 0))
                    for _ in range(3)]),
        out_shape=([jax.ShapeDtypeStruct((B, S, N // 4), jnp.int32)
                    for _ in range(3)] +
                   [jax.ShapeDtypeStruct((B, S, 1), jnp.int32)
                    for _ in range(3)]),
    )(new_xyz, xyz)
    return ([o.reshape(B * S, N // 4) for o in outs[:3]],
            [o.reshape(B * S) for o in outs[3:]])


def _sc_extract_gather(table, e_packed, stopw, K, N):
    RQ, NWRD = e_packed.shape
    NC, NS = 2, 16
    rows_per_w = RQ // (NC * NS)
    mesh = plsc.VectorSubcoreMesh(core_axis_name="c", subcore_axis_name="s")

    @functools.partial(
        pl.kernel, mesh=mesh,
        compiler_params=pltpu.CompilerParams(needs_layout_passes=False),
        out_type=jax.ShapeDtypeStruct((RQ * K, 128), jnp.float32),
        scratch_types=[
            pltpu.VMEM((2, NWRD), jnp.int32),
            pltpu.VMEM((8, 128), jnp.int32),
            pltpu.VMEM((K,), jnp.int32),
            pltpu.VMEM((2, K, 128), jnp.float32),
            pltpu.VMEM((rows_per_w,), jnp.int32),
            pltpu.SemaphoreType.DMA,
            pltpu.SemaphoreType.DMA,
            pltpu.SemaphoreType.DMA,
            pltpu.SemaphoreType.DMA,
            pltpu.SemaphoreType.DMA,
        ],
    )
    def k(table_hbm, e_hbm, stop_hbm, out_hbm, e_v, idx_v, gidx_v, rows_v,
          stop_v, sem_g, se0, se1, so0, so1):
        wid = lax.axis_index("s") * NC + lax.axis_index("c")
        base = wid * rows_per_w
        lane = jax.lax.iota(jnp.int32, 16)
        zero16 = jnp.zeros((16,), jnp.int32)
        sentinel = jnp.full((16,), N, jnp.int32)
        pltpu.sync_copy(stop_hbm.at[pl.ds(base, rows_per_w)], stop_v)
        NCH = NWRD // 16

        def cstop_of(i):
            sv = stop_v[pl.ds((i // 16) * 16, 16)]
            sj = jnp.max(jnp.where(lane == (i % 16), sv, -1), axis=0)
            return jnp.minimum(NCH, (sj >> 9) * 8 + 8)

        def prefetch(i, slot, sem):
            nw = cstop_of(i) * 16
            pltpu.async_copy(e_hbm.at[base + i, pl.ds(0, 512)],
                             e_v.at[slot, pl.ds(0, 512)], sem)
            @pl.when(nw > 512)
            def _():
                pltpu.async_copy(e_hbm.at[base + i, pl.ds(512, NWRD - 512)],
                                 e_v.at[slot, pl.ds(512, NWRD - 512)], sem)

        def wait_prefetch(i, slot, sem):
            nw = cstop_of(i) * 16
            pltpu.make_async_copy(e_hbm.at[base + i, pl.ds(0, 512)],
                                  e_v.at[slot, pl.ds(0, 512)], sem).wait()
            @pl.when(nw > 512)
            def _():
                pltpu.make_async_copy(
                    e_hbm.at[base + i, pl.ds(512, NWRD - 512)],
                    e_v.at[slot, pl.ds(512, NWRD - 512)], sem).wait()

        prefetch(0, 0, se0)

        def row_body(i, car):
            row = base + i
            slot = lax.rem(i, 2)
            esem = (se0, se1)
            osem = (so0, so1)

            @pl.when(i + 1 < rows_per_w)
            def _():
                @pl.when(slot == 0)
                def _():
                    prefetch(i + 1, 1, se1)
                @pl.when(slot == 1)
                def _():
                    prefetch(i + 1, 0, se0)

            @pl.when(slot == 0)
            def _():
                wait_prefetch(i, 0, se0)
            @pl.when(slot == 1)
            def _():
                wait_prefetch(i, 1, se1)

            for g in range(K // 16):
                idx_v[0, pl.ds(g * 16, 16)] = sentinel

            c_stop = cstop_of(i)

            def make_wchunk(slot_c):
                def wchunk(c, _c):
                    wv = e_v[slot_c, pl.ds(c * 16, 16)]
                    jb = 512 * (c // 8) + (c % 8) * 16
                    for b in range(4):
                        eb = (wv >> (8 * b)) & 0xFF
                        valid = (eb > 0) & (eb <= K)
                        plsc.store_scatter(idx_v, [zero16, eb - 1],
                                           lane + (jb + 128 * b), mask=valid)
                    return _c
                return wchunk

            @pl.when(slot == 0)
            def _():
                lax.fori_loop(0, c_stop, make_wchunk(0), 0)
            @pl.when(slot == 1)
            def _():
                lax.fori_loop(0, c_stop, make_wchunk(1), 0)

            first = jnp.broadcast_to(
                jnp.min(idx_v[0, pl.ds(0, 16)], axis=0), (16,))
            boff = jnp.full((16,), (row // _NPOINT) * N, jnp.int32)
            for g in range(K // 16):
                v = idx_v[0, pl.ds(g * 16, 16)]
                gidx_v[pl.ds(g * 16, 16)] = (
                    jnp.where(v == N, first, v) + boff)

            def do_slot(s):
                @pl.when(i >= 2)
                def _():
                    pltpu.make_async_copy(
                        rows_v.at[s], out_hbm.at[pl.ds(row * K, K)],
                        osem[s]).wait()
                pltpu.async_copy(table_hbm.at[gidx_v], rows_v.at[s],
                                 sem_g).wait()
                pltpu.async_copy(rows_v.at[s], out_hbm.at[pl.ds(row * K, K)],
                                 osem[s])

            @pl.when(slot == 0)
            def _():
                do_slot(0)
            @pl.when(slot == 1)
            def _():
                do_slot(1)
            return car

        lax.fori_loop(0, rows_per_w, row_body, 0)
        for s, i_last in ((rows_per_w % 2, rows_per_w - 2),
                          ((rows_per_w - 1) % 2, rows_per_w - 1)):
            pltpu.make_async_copy(
                rows_v.at[s],
                out_hbm.at[pl.ds((base + i_last) * K, K)],
                (so0, so1)[s]).wait()

    return k(table, e_packed, stopw)


def _fps_body(xyz_ref, out_ref, nxyz_ref):
    B, _, N = xyz_ref.shape
    x = xyz_ref[:, 0, :]
    y = xyz_ref[:, 1, :]
    z = xyz_ref[:, 2, :]
    iota = jax.lax.broadcasted_iota(jnp.int32, (B, N), 1)

    def step(i, carry):
        dist, far = carry
        out_ref[pl.ds(i, 1), :] = far.T
        sel = iota == far
        cx = jnp.sum(jnp.where(sel, x, 0.0), axis=1, keepdims=True)
        cy = jnp.sum(jnp.where(sel, y, 0.0), axis=1, keepdims=True)
        cz = jnp.sum(jnp.where(sel, z, 0.0), axis=1, keepdims=True)
        nxyz_ref[pl.ds(i, 1), :, :] = jnp.concatenate(
            [cx, cy, cz], axis=1)[None, :, :]
        dx = x - cx
        dy = y - cy
        dz = z - cz
        d = dx * dx + dy * dy + dz * dz
        dist = jnp.minimum(dist, d)
        m = jnp.max(dist, axis=1, keepdims=True)
        far_new = jnp.min(jnp.where(dist == m, iota, N), axis=1, keepdims=True)
        return dist, far_new.astype(jnp.int32)

    dist0 = jnp.full((B, N), 1e10, dtype=jnp.float32)
    far0 = jnp.zeros((B, 1), dtype=jnp.int32)
    jax.lax.fori_loop(0, out_ref.shape[0], step, (dist0, far0))


def _fps(xyz):
    B, _, N = xyz.shape
    out, nxyz = pl.pallas_call(
        _fps_body,
        out_shape=[jax.ShapeDtypeStruct((_NPOINT, B), jnp.int32),
                   jax.ShapeDtypeStruct((_NPOINT, B, 3), jnp.float32)],
        in_specs=[pl.BlockSpec(memory_space=pltpu.MemorySpace.VMEM)],
        out_specs=[pl.BlockSpec(memory_space=pltpu.MemorySpace.VMEM),
                   pl.BlockSpec(memory_space=pltpu.MemorySpace.VMEM)],
    )(xyz)
    return out.T, jnp.transpose(nxyz, (1, 0, 2))


def _index_points(points, idx):
    return jax.vmap(lambda p, i: p[i])(points, idx)


def kernel(xyz, points, params):
    B, _, N = xyz.shape
    S = _NPOINT
    xyz_t = jnp.transpose(xyz, (0, 2, 1))
    pts_t = jnp.transpose(points, (0, 2, 1))

    _, new_xyz = _fps(xyz)

    table = jnp.concatenate(
        [pts_t, xyz_t, jnp.zeros((B, N, _TABLE_D - 67), jnp.float32)],
        axis=-1).reshape(B * N, _TABLE_D)

    e_packed, stops = _select(new_xyz, xyz)

    outs = []
    for ri, (r, K) in enumerate(zip(_RADII, _NSAMPLES)):
        g_rows = _sc_extract_gather(table, e_packed[ri], stops[ri], K, N)
        g_rows = g_rows.reshape(B, S, K, _TABLE_D)
        g_pts = g_rows[..., :64]
        g_xyz = g_rows[..., 64:67] - new_xyz[:, :, None, :]
        g = jnp.concatenate([g_pts, g_xyz], axis=-1)
        g = jnp.transpose(g, (0, 3, 2, 1))
        for layer in params[len(outs)]:
            g = jnp.einsum('oc,bcks->boks', layer["W"], g) + layer["b"][None, :, None, None]
            mean = jnp.mean(g, axis=(0, 2, 3), keepdims=True)
            var = jnp.var(g, axis=(0, 2, 3), keepdims=True)
            g = (g - mean) / jnp.sqrt(var + 1e-5)
            g = g * layer["gamma"][None, :, None, None] + layer["beta"][None, :, None, None]
            g = jax.nn.relu(g)
        outs.append(jnp.max(g, axis=2))

    return (jnp.transpose(new_xyz, (0, 2, 1)), jnp.concatenate(outs, axis=1))


# speedup vs baseline: 1.2153x; 1.0838x over previous
import functools

import jax
import jax.numpy as jnp
from jax import lax
from jax.experimental import pallas as pl
from jax.experimental.pallas import tpu as pltpu
from jax.experimental.pallas import tpu_sc as plsc

_NPOINT = 512
_RADII = (0.1, 0.2, 0.4)
_NSAMPLES = (16, 32, 64)
_TABLE_D = 128


_SBLK = 128


def _select_body(q_ref, p_ref, e1_ref, e2_ref, e3_ref,
                 s1_ref, s2_ref, s3_ref):
    N = p_ref.shape[2]
    q = q_ref[0]
    p = p_ref[0]
    qp = jax.lax.dot_general(q, p, (((1,), (0,)), ((), ())),
                             preferred_element_type=jnp.float32)
    q2 = jnp.sum(q * q, axis=1, keepdims=True)
    p2 = jnp.sum(p * p, axis=0, keepdims=True)
    d = -2.0 * qp + q2 + p2

    masks = [(d <= r * r).astype(jnp.float32) for r in _RADII]
    m_all = jnp.concatenate(masks, axis=0)

    it = jax.lax.broadcasted_iota(jnp.int32, (128, 128), 0)
    jt = jax.lax.broadcasted_iota(jnp.int32, (128, 128), 1)
    T = (it <= jt).astype(jnp.float32)

    carry = jnp.zeros((3 * _SBLK, 1), jnp.float32)
    smax = [jnp.full((_SBLK, 1), -1, jnp.int32) for _ in range(3)]
    jl0 = jax.lax.broadcasted_iota(jnp.int32, (_SBLK, 128), 1)
    for g in range(N // 128):
        seg = m_all[:, g * 128:(g + 1) * 128]
        loc = jax.lax.dot_general(seg, T, (((1,), (0,)), ((), ())),
                                  preferred_element_type=jnp.float32)
        C = loc + carry
        carry = carry + loc[:, 127:128]
        byte = g % 4
        woff = (g // 4) * 128
        for ri, (e_ref, K) in enumerate(
                zip((e1_ref, e2_ref, e3_ref), _NSAMPLES)):
            Cg = C[ri * _SBLK:(ri + 1) * _SBLK]
            mg = m_all[ri * _SBLK:(ri + 1) * _SBLK, g * 128:(g + 1) * 128]
            e = jnp.where(mg > 0.0,
                          jnp.minimum(Cg, float(K + 1)), 0.0).astype(jnp.int32)
            cand = jnp.max(jnp.where(e == K, jl0 + g * 128, -1),
                           axis=1, keepdims=True)
            smax[ri] = jnp.maximum(smax[ri], cand)
            word = e << (8 * byte)
            if byte == 0:
                e_ref[0, :, woff:woff + 128] = word
            else:
                e_ref[0, :, woff:woff + 128] = e_ref[0, :, woff:woff + 128] | word
    for ri, s_ref in enumerate((s1_ref, s2_ref, s3_ref)):
        s_ref[0] = jnp.where(smax[ri] < 0, N, smax[ri])


def _select(new_xyz, xyz):
    B, _, N = xyz.shape
    S = _NPOINT
    grid = (B, S // _SBLK)
    outs = pl.pallas_call(
        _select_body,
        grid=grid,
        in_specs=[
            pl.BlockSpec((1, _SBLK, 3), lambda b, s: (b, s, 0)),
            pl.BlockSpec((1, 3, N), lambda b, s: (b, 0, 0)),
        ],
        out_specs=([pl.BlockSpec((1, _SBLK, N // 4), lambda b, s: (b, s, 0))
                    for _ in range(3)] +
                   [pl.BlockSpec((1, _SBLK, 1), lambda b, s: (b, s, 0))
                    for _ in range(3)]),
        out_shape=([jax.ShapeDtypeStruct((B, S, N // 4), jnp.int32)
                    for _ in range(3)] +
                   [jax.ShapeDtypeStruct((B, S, 1), jnp.int32)
                    for _ in range(3)]),
    )(new_xyz, xyz)
    return ([o.reshape(B * S, N // 4) for o in outs[:3]],
            [o.reshape(B * S) for o in outs[3:]])


def _sc_extract_gather(table, e_packed, stopw, K, N):
    RQ, NWRD = e_packed.shape
    NC, NS = 2, 16
    rows_per_w = RQ // (NC * NS)
    mesh = plsc.VectorSubcoreMesh(core_axis_name="c", subcore_axis_name="s")

    @functools.partial(
        pl.kernel, mesh=mesh,
        compiler_params=pltpu.CompilerParams(needs_layout_passes=False),
        out_type=jax.ShapeDtypeStruct((RQ * K, 128), jnp.float32),
        scratch_types=[
            pltpu.VMEM((2, NWRD), jnp.int32),
            pltpu.VMEM((8, 128), jnp.int32),
            pltpu.VMEM((2, K), jnp.int32),
            pltpu.VMEM((2, K, 128), jnp.float32),
            pltpu.VMEM((rows_per_w,), jnp.int32),
            pltpu.SemaphoreType.DMA,
            pltpu.SemaphoreType.DMA,
            pltpu.SemaphoreType.DMA,
            pltpu.SemaphoreType.DMA,
            pltpu.SemaphoreType.DMA,
            pltpu.SemaphoreType.DMA,
        ],
    )
    def k(table_hbm, e_hbm, stop_hbm, out_hbm, e_v, idx_v, gidx_v, rows_v,
          stop_v, sg0, sg1, se0, se1, so0, so1):
        wid = lax.axis_index("s") * NC + lax.axis_index("c")
        base = wid * rows_per_w
        lane = jax.lax.iota(jnp.int32, 16)
        zero16 = jnp.zeros((16,), jnp.int32)
        sentinel = jnp.full((16,), N, jnp.int32)
        pltpu.sync_copy(stop_hbm.at[pl.ds(base, rows_per_w)], stop_v)
        NCH = NWRD // 16

        def cstop_of(i):
            sv = stop_v[pl.ds((i // 16) * 16, 16)]
            sj = jnp.max(jnp.where(lane == (i % 16), sv, -1), axis=0)
            return jnp.minimum(NCH, (sj >> 9) * 8 + 8)

        def prefetch(i, slot, sem):
            nw = cstop_of(i) * 16
            pltpu.async_copy(e_hbm.at[base + i, pl.ds(0, 512)],
                             e_v.at[slot, pl.ds(0, 512)], sem)
            @pl.when(nw > 512)
            def _():
                pltpu.async_copy(e_hbm.at[base + i, pl.ds(512, NWRD - 512)],
                                 e_v.at[slot, pl.ds(512, NWRD - 512)], sem)

        def wait_prefetch(i, slot, sem):
            nw = cstop_of(i) * 16
            pltpu.make_async_copy(e_hbm.at[base + i, pl.ds(0, 512)],
                                  e_v.at[slot, pl.ds(0, 512)], sem).wait()
            @pl.when(nw > 512)
            def _():
                pltpu.make_async_copy(
                    e_hbm.at[base + i, pl.ds(512, NWRD - 512)],
                    e_v.at[slot, pl.ds(512, NWRD - 512)], sem).wait()

        prefetch(0, 0, se0)

        osem = (so0, so1)
        gsem = (sg0, sg1)

        def row_body(i, car):
            row = base + i
            slot = lax.rem(i, 2)

            @pl.when(i + 1 < rows_per_w)
            def _():
                @pl.when(slot == 0)
                def _():
                    prefetch(i + 1, 1, se1)
                @pl.when(slot == 1)
                def _():
                    prefetch(i + 1, 0, se0)

            @pl.when(slot == 0)
            def _():
                wait_prefetch(i, 0, se0)
            @pl.when(slot == 1)
            def _():
                wait_prefetch(i, 1, se1)

            for g in range(K // 16):
                idx_v[0, pl.ds(g * 16, 16)] = sentinel

            c_stop = cstop_of(i)

            def make_wchunk(slot_c):
                def wchunk(c, _c):
                    wv = e_v[slot_c, pl.ds(c * 16, 16)]
                    jb = 512 * (c // 8) + (c % 8) * 16
                    for b in range(4):
                        eb = (wv >> (8 * b)) & 0xFF
                        valid = (eb > 0) & (eb <= K)
                        plsc.store_scatter(idx_v, [zero16, eb - 1],
                                           lane + (jb + 128 * b), mask=valid)
                    return _c
                return wchunk

            @pl.when(slot == 0)
            def _():
                lax.fori_loop(0, c_stop, make_wchunk(0), 0)
            @pl.when(slot == 1)
            def _():
                lax.fori_loop(0, c_stop, make_wchunk(1), 0)

            first = jnp.broadcast_to(
                jnp.min(idx_v[0, pl.ds(0, 16)], axis=0), (16,))
            boff = jnp.full((16,), (row // _NPOINT) * N, jnp.int32)
            for g in range(K // 16):
                v = idx_v[0, pl.ds(g * 16, 16)]
                gv = jnp.where(v == N, first, v) + boff
                @pl.when(slot == 0)
                def _(gv=gv, g=g):
                    gidx_v[0, pl.ds(g * 16, 16)] = gv
                @pl.when(slot == 1)
                def _(gv=gv, g=g):
                    gidx_v[1, pl.ds(g * 16, 16)] = gv

            def do_slot(s):
                @pl.when(i >= 2)
                def _():
                    pltpu.make_async_copy(
                        rows_v.at[s], out_hbm.at[pl.ds(row * K, K)],
                        osem[s]).wait()
                pltpu.async_copy(table_hbm.at[gidx_v.at[s]], rows_v.at[s],
                                 gsem[s])
                @pl.when(i >= 1)
                def _():
                    pltpu.make_async_copy(
                        table_hbm.at[gidx_v.at[1 - s]], rows_v.at[1 - s],
                        gsem[1 - s]).wait()
                    pltpu.async_copy(
                        rows_v.at[1 - s],
                        out_hbm.at[pl.ds((row - 1) * K, K)], osem[1 - s])

            @pl.when(slot == 0)
            def _():
                do_slot(0)
            @pl.when(slot == 1)
            def _():
                do_slot(1)
            return car

        lax.fori_loop(0, rows_per_w, row_body, 0)
        s_last = (rows_per_w - 1) % 2
        i_last = rows_per_w - 1
        pltpu.make_async_copy(
            table_hbm.at[gidx_v.at[s_last]], rows_v.at[s_last],
            gsem[s_last]).wait()
        pltpu.async_copy(
            rows_v.at[s_last],
            out_hbm.at[pl.ds((base + i_last) * K, K)], osem[s_last])
        for s, i_d in ((rows_per_w % 2, rows_per_w - 2),
                       (s_last, i_last)):
            pltpu.make_async_copy(
                rows_v.at[s],
                out_hbm.at[pl.ds((base + i_d) * K, K)],
                osem[s]).wait()

    return k(table, e_packed, stopw)


def _fps_body(xyz_ref, out_ref, nxyz_ref):
    B, _, N = xyz_ref.shape
    x = xyz_ref[:, 0, :]
    y = xyz_ref[:, 1, :]
    z = xyz_ref[:, 2, :]
    iota = jax.lax.broadcasted_iota(jnp.int32, (B, N), 1)

    def step(i, carry):
        dist, far = carry
        out_ref[pl.ds(i, 1), :] = far.T
        sel = iota == far
        cx = jnp.sum(jnp.where(sel, x, 0.0), axis=1, keepdims=True)
        cy = jnp.sum(jnp.where(sel, y, 0.0), axis=1, keepdims=True)
        cz = jnp.sum(jnp.where(sel, z, 0.0), axis=1, keepdims=True)
        nxyz_ref[pl.ds(i, 1), :, :] = jnp.concatenate(
            [cx, cy, cz], axis=1)[None, :, :]
        dx = x - cx
        dy = y - cy
        dz = z - cz
        d = dx * dx + dy * dy + dz * dz
        dist = jnp.minimum(dist, d)
        m = jnp.max(dist, axis=1, keepdims=True)
        far_new = jnp.min(jnp.where(dist == m, iota, N), axis=1, keepdims=True)
        return dist, far_new.astype(jnp.int32)

    dist0 = jnp.full((B, N), 1e10, dtype=jnp.float32)
    far0 = jnp.zeros((B, 1), dtype=jnp.int32)
    jax.lax.fori_loop(0, out_ref.shape[0], step, (dist0, far0))


def _fps(xyz):
    B, _, N = xyz.shape
    out, nxyz = pl.pallas_call(
        _fps_body,
        out_shape=[jax.ShapeDtypeStruct((_NPOINT, B), jnp.int32),
                   jax.ShapeDtypeStruct((_NPOINT, B, 3), jnp.float32)],
        in_specs=[pl.BlockSpec(memory_space=pltpu.MemorySpace.VMEM)],
        out_specs=[pl.BlockSpec(memory_space=pltpu.MemorySpace.VMEM),
                   pl.BlockSpec(memory_space=pltpu.MemorySpace.VMEM)],
    )(xyz)
    return out.T, jnp.transpose(nxyz, (1, 0, 2))


def _index_points(points, idx):
    return jax.vmap(lambda p, i: p[i])(points, idx)


def kernel(xyz, points, params):
    B, _, N = xyz.shape
    S = _NPOINT
    xyz_t = jnp.transpose(xyz, (0, 2, 1))
    pts_t = jnp.transpose(points, (0, 2, 1))

    _, new_xyz = _fps(xyz)

    table = jnp.concatenate(
        [pts_t, xyz_t, jnp.zeros((B, N, _TABLE_D - 67), jnp.float32)],
        axis=-1).reshape(B * N, _TABLE_D)

    e_packed, stops = _select(new_xyz, xyz)

    outs = []
    for ri, (r, K) in enumerate(zip(_RADII, _NSAMPLES)):
        g_rows = _sc_extract_gather(table, e_packed[ri], stops[ri], K, N)
        g_rows = g_rows.reshape(B, S, K, _TABLE_D)
        g_pts = g_rows[..., :64]
        g_xyz = g_rows[..., 64:67] - new_xyz[:, :, None, :]
        g = jnp.concatenate([g_pts, g_xyz], axis=-1)
        g = jnp.transpose(g, (0, 3, 2, 1))
        for layer in params[len(outs)]:
            g = jnp.einsum('oc,bcks->boks', layer["W"], g) + layer["b"][None, :, None, None]
            mean = jnp.mean(g, axis=(0, 2, 3), keepdims=True)
            var = jnp.var(g, axis=(0, 2, 3), keepdims=True)
            g = (g - mean) / jnp.sqrt(var + 1e-5)
            g = g * layer["gamma"][None, :, None, None] + layer["beta"][None, :, None, None]
            g = jax.nn.relu(g)
        outs.append(jnp.max(g, axis=2))

    return (jnp.transpose(new_xyz, (0, 2, 1)), jnp.concatenate(outs, axis=1))
